# Initial kernel scaffold; baseline (speedup 1.0000x reference)
import jax
import jax.numpy as jnp
from jax.experimental import pallas as pl


def kernel(word_table, pos1_table, pos2_table, W, b, word, h_entity_word, t_entity_word, pos1, pos2):
    raise NotImplementedError("write your pallas kernel here")


# speedup vs baseline: 2.2378x; 2.2378x over previous
import jax
import jax.numpy as jnp
from jax import lax
from jax.experimental import pallas as pl
from jax.experimental.pallas import tpu as pltpu
from jax.experimental.pallas import tpu_sc as plsc

B, L = 1024, 120
V, D = 100000, 50
P, PD = 512, 5
N = B * L
NC, NS = 2, 16
NW = NC * NS
PER_W = N // NW
CHUNK = 640
NCHUNK = PER_W // CHUNK
EPW = B // NW

GD = 55
K = 8
R = K * L


def _sc_gather_body(table, widx, hidx, tidx, we_out, he_out, te_out,
                    idx_v, rows_v, eidx_v, erows_v, sem):
  wid = lax.axis_index("s") * NC + lax.axis_index("c")
  base = wid * PER_W
  for c in range(NCHUNK):
    off = base + c * CHUNK
    pltpu.sync_copy(widx.at[pl.ds(off, CHUNK)], idx_v)
    pltpu.async_copy(table.at[idx_v], rows_v, sem).wait()
    pltpu.sync_copy(rows_v, we_out.at[pl.ds(off, CHUNK)])
  ebase = wid * EPW
  pltpu.sync_copy(hidx.at[pl.ds(ebase, EPW)], eidx_v)
  pltpu.async_copy(table.at[eidx_v], erows_v, sem).wait()
  pltpu.sync_copy(erows_v, he_out.at[pl.ds(ebase, EPW)])
  pltpu.sync_copy(tidx.at[pl.ds(ebase, EPW)], eidx_v)
  pltpu.async_copy(table.at[eidx_v], erows_v, sem).wait()
  pltpu.sync_copy(erows_v, te_out.at[pl.ds(ebase, EPW)])


def _make_sc_gather(interpret=False):
  mesh = plsc.VectorSubcoreMesh(core_axis_name="c", subcore_axis_name="s",
                                num_cores=NC, num_subcores=NS)
  return pl.kernel(
      _sc_gather_body,
      out_type=[jax.ShapeDtypeStruct((N, D), jnp.float32),
                jax.ShapeDtypeStruct((B, D), jnp.float32),
                jax.ShapeDtypeStruct((B, D), jnp.float32)],
      mesh=mesh,
      scratch_types=[pltpu.VMEM((CHUNK,), jnp.int32),
                     pltpu.VMEM((CHUNK, D), jnp.float32),
                     pltpu.VMEM((EPW,), jnp.int32),
                     pltpu.VMEM((EPW, D), jnp.float32),
                     pltpu.SemaphoreType.DMA],
      compiler_params=pltpu.CompilerParams(use_tc_tiling_on_sc=False),
      interpret=interpret,
  )


def _dense_body(we_ref, he_ref, te_ref, p1_ref, p2_ref,
                tab1_ref, tab2_ref, wa_ref, wb_ref, w3_ref, bs_ref, out_ref):
  we = we_ref[...].reshape(R, D)
  he = he_ref[...]
  te = te_ref[...]
  p1 = p1_ref[...]
  p2 = p2_ref[...]
  iot = lax.broadcasted_iota(jnp.int32, (1, P), 1)
  oh1 = (p1 == iot).astype(jnp.float32)
  oh2 = (p2 == iot).astype(jnp.float32)
  tab1 = tab1_ref[...]
  tab2 = tab2_ref[...]
  w3 = w3_ref[...]
  t1w = lax.dot_general(tab1, w3, (((1,), (1,)), ((), ())),
                        preferred_element_type=jnp.float32)
  cat1 = jnp.concatenate([tab1, t1w], axis=1)
  r1 = lax.dot_general(oh1, cat1, (((1,), (0,)), ((), ())),
                       preferred_element_type=jnp.float32)
  p1r = r1[:, 0:PD]
  p1l = r1[:, PD:PD + GD]
  p2r = lax.dot_general(oh2, tab2, (((1,), (0,)), ((), ())),
                        preferred_element_type=jnp.float32)
  wl = lax.dot_general(we, wa_ref[...], (((1,), (1,)), ((), ())),
                       preferred_element_type=jnp.float32)
  hl = lax.dot_general(he, wb_ref[...], (((1,), (1,)), ((), ())),
                       preferred_element_type=jnp.float32)
  hl = hl + bs_ref[...]
  hlb = jnp.broadcast_to(hl[:, None, :], (K, L, GD)).reshape(R, GD)
  a = 1.0 / (1.0 + jnp.exp(-(wl + p1l + hlb)))
  amid = a[:, 0:D]
  ahi = a[:, D:GD]
  heb = jnp.broadcast_to(he[:, None, :], (K, L, D)).reshape(R, D)
  teb = jnp.broadcast_to(te[:, None, :], (K, L, D)).reshape(R, D)
  mid = teb + amid * (heb - teb)
  hi = p2r + ahi * (p1r - p2r)
  out = jnp.concatenate([we, mid, hi], axis=1)
  out_ref[...] = out.reshape(K, L, 105)


def _make_tc_dense(interpret=False):
  return pl.pallas_call(
      _dense_body,
      grid=(B // K,),
      in_specs=[
          pl.BlockSpec((K, L, D), lambda i: (i, 0, 0)),
          pl.BlockSpec((K, D), lambda i: (i, 0)),
          pl.BlockSpec((K, D), lambda i: (i, 0)),
          pl.BlockSpec((R, 1), lambda i: (i, 0)),
          pl.BlockSpec((R, 1), lambda i: (i, 0)),
          pl.BlockSpec((P, PD), lambda i: (0, 0)),
          pl.BlockSpec((P, PD), lambda i: (0, 0)),
          pl.BlockSpec((GD, D), lambda i: (0, 0)),
          pl.BlockSpec((GD, D), lambda i: (0, 0)),
          pl.BlockSpec((GD, PD), lambda i: (0, 0)),
          pl.BlockSpec((K, GD), lambda i: (0, 0)),
      ],
      out_specs=pl.BlockSpec((K, L, 105), lambda i: (i, 0, 0)),
      out_shape=jax.ShapeDtypeStruct((B, L, 105), jnp.float32),
      interpret=interpret,
  )


def kernel(word_table, pos1_table, pos2_table, W, b, word,
           h_entity_word, t_entity_word, pos1, pos2):
  widx = word.reshape(N).astype(jnp.int32)
  hidx = h_entity_word.reshape(B).astype(jnp.int32)
  tidx = t_entity_word.reshape(B).astype(jnp.int32)
  we, he, te = _make_sc_gather()(word_table, widx, hidx, tidx)
  wa = W[50:105, 0:50]
  wb = W[50:105, 50:100]
  w3 = W[50:105, 100:105]
  bs = jnp.broadcast_to(b[50:105], (K, GD))
  return _make_tc_dense()(we.reshape(B, L, D), he, te,
                          pos1.reshape(N, 1).astype(jnp.int32),
                          pos2.reshape(N, 1).astype(jnp.int32),
                          pos1_table, pos2_table, wa, wb, w3, bs)


# speedup vs baseline: 2.6270x; 1.1739x over previous
import jax
import jax.numpy as jnp
from jax import lax
from jax.experimental import pallas as pl
from jax.experimental.pallas import tpu as pltpu
from jax.experimental.pallas import tpu_sc as plsc

B, L = 1024, 120
V, D = 100000, 50
P, PD = 512, 5
WD = 128
PDW = 8
N = B * L
NC, NS = 2, 16
NW = NC * NS
PER_W = N // NW
CHUNK = 480
NCHUNK = PER_W // CHUNK
EPW = B // NW

GD = 55
K = 8
R = K * L


def _sc_gather_body(table, widx, hidx, tidx, we_out, he_out, te_out,
                    idx_v, rows_v, eidx_v, erows_v, sem):
  wid = lax.axis_index("s") * NC + lax.axis_index("c")
  base = wid * PER_W
  for c in range(NCHUNK):
    off = base + c * CHUNK
    pltpu.sync_copy(widx.at[pl.ds(off, CHUNK)], idx_v)
    pltpu.async_copy(table.at[idx_v], rows_v, sem).wait()
    pltpu.sync_copy(rows_v, we_out.at[pl.ds(off, CHUNK)])
  ebase = wid * EPW
  pltpu.sync_copy(hidx.at[pl.ds(ebase, EPW)], eidx_v)
  pltpu.async_copy(table.at[eidx_v], erows_v, sem).wait()
  pltpu.sync_copy(erows_v, he_out.at[pl.ds(ebase, EPW)])
  pltpu.sync_copy(tidx.at[pl.ds(ebase, EPW)], eidx_v)
  pltpu.async_copy(table.at[eidx_v], erows_v, sem).wait()
  pltpu.sync_copy(erows_v, te_out.at[pl.ds(ebase, EPW)])


def _make_sc_gather(interpret=False):
  mesh = plsc.VectorSubcoreMesh(core_axis_name="c", subcore_axis_name="s",
                                num_cores=NC, num_subcores=NS)
  return pl.kernel(
      _sc_gather_body,
      out_type=[jax.ShapeDtypeStruct((N, WD), jnp.float32),
                jax.ShapeDtypeStruct((B, WD), jnp.float32),
                jax.ShapeDtypeStruct((B, WD), jnp.float32)],
      mesh=mesh,
      scratch_types=[pltpu.VMEM((CHUNK,), jnp.int32),
                     pltpu.VMEM((CHUNK, WD), jnp.float32),
                     pltpu.VMEM((EPW,), jnp.int32),
                     pltpu.VMEM((EPW, WD), jnp.float32),
                     pltpu.SemaphoreType.DMA],
      compiler_params=pltpu.CompilerParams(use_tc_tiling_on_sc=False),
      interpret=interpret,
  )


def _sc_pos_body(ptab1, ptab2, p1idx, p2idx, p1_out, p2_out,
                 pidx_v, prow_v, sem):
  wid = lax.axis_index("s") * NC + lax.axis_index("c")
  base = wid * PER_W
  pltpu.sync_copy(p1idx.at[pl.ds(base, PER_W)], pidx_v)
  pltpu.async_copy(ptab1.at[pidx_v], prow_v, sem).wait()
  pltpu.sync_copy(prow_v, p1_out.at[pl.ds(base, PER_W)])
  pltpu.sync_copy(p2idx.at[pl.ds(base, PER_W)], pidx_v)
  pltpu.async_copy(ptab2.at[pidx_v], prow_v, sem).wait()
  pltpu.sync_copy(prow_v, p2_out.at[pl.ds(base, PER_W)])


def _make_sc_pos(interpret=False):
  mesh = plsc.VectorSubcoreMesh(core_axis_name="c", subcore_axis_name="s",
                                num_cores=NC, num_subcores=NS)
  return pl.kernel(
      _sc_pos_body,
      out_type=[jax.ShapeDtypeStruct((N, PDW), jnp.float32),
                jax.ShapeDtypeStruct((N, PDW), jnp.float32)],
      mesh=mesh,
      scratch_types=[pltpu.VMEM((PER_W,), jnp.int32),
                     pltpu.VMEM((PER_W, PDW), jnp.float32),
                     pltpu.SemaphoreType.DMA],
      compiler_params=pltpu.CompilerParams(use_tc_tiling_on_sc=False),
      interpret=interpret,
  )


def _dense_body(we_ref, he_ref, te_ref, p1_ref, p2_ref,
                wa_ref, wb_ref, w3_ref, bs_ref, out_ref):
  we = we_ref[...][:, :, 0:D].reshape(R, D)
  he = he_ref[...][:, 0:D]
  te = te_ref[...][:, 0:D]
  p1r = p1_ref[...].reshape(R, PDW)
  p2r = p2_ref[...].reshape(R, PDW)
  wl = lax.dot_general(we, wa_ref[...], (((1,), (1,)), ((), ())),
                       preferred_element_type=jnp.float32)
  p1l = lax.dot_general(p1r, w3_ref[...], (((1,), (1,)), ((), ())),
                        preferred_element_type=jnp.float32)
  hl = lax.dot_general(he, wb_ref[...], (((1,), (1,)), ((), ())),
                       preferred_element_type=jnp.float32)
  hl = hl + bs_ref[...]
  hlb = jnp.broadcast_to(hl[:, None, :], (K, L, GD)).reshape(R, GD)
  a = 1.0 / (1.0 + jnp.exp(-(wl + p1l + hlb)))
  amid = a[:, 0:D]
  ahi = a[:, D:GD]
  heb = jnp.broadcast_to(he[:, None, :], (K, L, D)).reshape(R, D)
  teb = jnp.broadcast_to(te[:, None, :], (K, L, D)).reshape(R, D)
  mid = teb + amid * (heb - teb)
  p1v = p1r[:, 0:PD]
  p2v = p2r[:, 0:PD]
  hi = p2v + ahi * (p1v - p2v)
  out = jnp.concatenate([we, mid, hi], axis=1)
  out_ref[...] = out.reshape(K, L, 105)


def _make_tc_dense(interpret=False):
  return pl.pallas_call(
      _dense_body,
      grid=(B // K,),
      in_specs=[
          pl.BlockSpec((K, L, WD), lambda i: (i, 0, 0)),
          pl.BlockSpec((K, WD), lambda i: (i, 0)),
          pl.BlockSpec((K, WD), lambda i: (i, 0)),
          pl.BlockSpec((K, L, PDW), lambda i: (i, 0, 0)),
          pl.BlockSpec((K, L, PDW), lambda i: (i, 0, 0)),
          pl.BlockSpec((GD, D), lambda i: (0, 0)),
          pl.BlockSpec((GD, D), lambda i: (0, 0)),
          pl.BlockSpec((GD, PDW), lambda i: (0, 0)),
          pl.BlockSpec((K, GD), lambda i: (0, 0)),
      ],
      out_specs=pl.BlockSpec((K, L, 105), lambda i: (i, 0, 0)),
      out_shape=jax.ShapeDtypeStruct((B, L, 105), jnp.float32),
      interpret=interpret,
  )


def kernel(word_table, pos1_table, pos2_table, W, b, word,
           h_entity_word, t_entity_word, pos1, pos2):
  widx = word.reshape(N).astype(jnp.int32)
  hidx = h_entity_word.reshape(B).astype(jnp.int32)
  tidx = t_entity_word.reshape(B).astype(jnp.int32)
  p1idx = pos1.reshape(N).astype(jnp.int32)
  p2idx = pos2.reshape(N).astype(jnp.int32)
  wt = jnp.pad(word_table, ((0, 0), (0, WD - D)))
  ptab1 = jnp.pad(pos1_table, ((0, 0), (0, PDW - PD)))
  ptab2 = jnp.pad(pos2_table, ((0, 0), (0, PDW - PD)))
  we, he, te = _make_sc_gather()(wt, widx, hidx, tidx)
  p1r, p2r = _make_sc_pos()(ptab1, ptab2, p1idx, p2idx)
  wa = W[50:105, 0:50]
  wb = W[50:105, 50:100]
  w3 = jnp.pad(W[50:105, 100:105], ((0, 0), (0, PDW - PD)))
  bs = jnp.broadcast_to(b[50:105], (K, GD))
  return _make_tc_dense()(we.reshape(B, L, WD), he, te,
                          p1r.reshape(B, L, PDW), p2r.reshape(B, L, PDW),
                          wa, wb, w3, bs)


# speedup vs baseline: 3.1031x; 1.1813x over previous
import jax
import jax.numpy as jnp
from jax import lax
from jax.experimental import pallas as pl
from jax.experimental.pallas import tpu as pltpu
from jax.experimental.pallas import tpu_sc as plsc

B, L = 1024, 120
V, D = 100000, 50
P, PD = 512, 5
WD = 128
N = B * L
NC, NS = 2, 16
NW = NC * NS
PER_W = N // NW
CHUNK = 480
NCHUNK = PER_W // CHUNK
EPW = B // NW

GD = 55
K = 8
R = K * L


def _sc_gather_body(table, widx, hidx, tidx, we_out, he_out, te_out,
                    idx_v, rows_v, eidx_v, erows_v, sem):
  wid = lax.axis_index("s") * NC + lax.axis_index("c")
  base = wid * PER_W
  for c in range(NCHUNK):
    off = base + c * CHUNK
    pltpu.sync_copy(widx.at[pl.ds(off, CHUNK)], idx_v)
    pltpu.async_copy(table.at[idx_v], rows_v, sem).wait()
    pltpu.sync_copy(rows_v, we_out.at[pl.ds(off, CHUNK)])
  ebase = wid * EPW
  pltpu.sync_copy(hidx.at[pl.ds(ebase, EPW)], eidx_v)
  pltpu.async_copy(table.at[eidx_v], erows_v, sem).wait()
  pltpu.sync_copy(erows_v, he_out.at[pl.ds(ebase, EPW)])
  pltpu.sync_copy(tidx.at[pl.ds(ebase, EPW)], eidx_v)
  pltpu.async_copy(table.at[eidx_v], erows_v, sem).wait()
  pltpu.sync_copy(erows_v, te_out.at[pl.ds(ebase, EPW)])


def _make_sc_gather(interpret=False):
  mesh = plsc.VectorSubcoreMesh(core_axis_name="c", subcore_axis_name="s",
                                num_cores=NC, num_subcores=NS)
  return pl.kernel(
      _sc_gather_body,
      out_type=[jax.ShapeDtypeStruct((N, WD), jnp.float32),
                jax.ShapeDtypeStruct((B, WD), jnp.float32),
                jax.ShapeDtypeStruct((B, WD), jnp.float32)],
      mesh=mesh,
      scratch_types=[pltpu.VMEM((CHUNK,), jnp.int32),
                     pltpu.VMEM((CHUNK, WD), jnp.float32),
                     pltpu.VMEM((EPW,), jnp.int32),
                     pltpu.VMEM((EPW, WD), jnp.float32),
                     pltpu.SemaphoreType.DMA],
      compiler_params=pltpu.CompilerParams(use_tc_tiling_on_sc=True),
      interpret=interpret,
  )


def _dense_body(we_ref, he_ref, te_ref, p1_ref, p2_ref,
                tab1_ref, tab2_ref, wa_ref, wb_ref, w3_ref, bs_ref, out_ref):
  we = we_ref[...][:, 0:D]
  he = he_ref[...][:, 0:D]
  te = te_ref[...][:, 0:D]
  p1 = p1_ref[...][:, :, None]
  p2 = p2_ref[...][:, :, None]
  iot = lax.broadcasted_iota(jnp.int32, (1, 1, P), 2)
  oh1 = (p1 == iot).astype(jnp.float32).reshape(R, P)
  oh2 = (p2 == iot).astype(jnp.float32).reshape(R, P)
  tab1 = tab1_ref[...]
  tab2 = tab2_ref[...]
  w3 = w3_ref[...]
  t1w = lax.dot_general(tab1, w3, (((1,), (1,)), ((), ())),
                        preferred_element_type=jnp.float32)
  cat1 = jnp.concatenate([tab1, t1w], axis=1)
  r1 = lax.dot_general(oh1, cat1, (((1,), (0,)), ((), ())),
                       preferred_element_type=jnp.float32)
  p1v = r1[:, 0:PD]
  p1l = r1[:, PD:PD + GD]
  p2v = lax.dot_general(oh2, tab2, (((1,), (0,)), ((), ())),
                        preferred_element_type=jnp.float32)
  wl = lax.dot_general(we, wa_ref[...], (((1,), (1,)), ((), ())),
                       preferred_element_type=jnp.float32)
  hl = lax.dot_general(he, wb_ref[...], (((1,), (1,)), ((), ())),
                       preferred_element_type=jnp.float32)
  hl = hl + bs_ref[...]
  hlb = jnp.broadcast_to(hl[:, None, :], (K, L, GD)).reshape(R, GD)
  a = 1.0 / (1.0 + jnp.exp(-(wl + p1l + hlb)))
  amid = a[:, 0:D]
  ahi = a[:, D:GD]
  heb = jnp.broadcast_to(he[:, None, :], (K, L, D)).reshape(R, D)
  teb = jnp.broadcast_to(te[:, None, :], (K, L, D)).reshape(R, D)
  mid = teb + amid * (heb - teb)
  hi = p2v + ahi * (p1v - p2v)
  out = jnp.concatenate([we, mid, hi], axis=1)
  out_ref[...] = out.reshape(K, L, 105)


def _make_tc_dense(interpret=False):
  return pl.pallas_call(
      _dense_body,
      grid=(B // K,),
      in_specs=[
          pl.BlockSpec((R, WD), lambda i: (i, 0)),
          pl.BlockSpec((K, WD), lambda i: (i, 0)),
          pl.BlockSpec((K, WD), lambda i: (i, 0)),
          pl.BlockSpec((K, L), lambda i: (i, 0)),
          pl.BlockSpec((K, L), lambda i: (i, 0)),
          pl.BlockSpec((P, PD), lambda i: (0, 0)),
          pl.BlockSpec((P, PD), lambda i: (0, 0)),
          pl.BlockSpec((GD, D), lambda i: (0, 0)),
          pl.BlockSpec((GD, D), lambda i: (0, 0)),
          pl.BlockSpec((GD, PD), lambda i: (0, 0)),
          pl.BlockSpec((K, GD), lambda i: (0, 0)),
      ],
      out_specs=pl.BlockSpec((K, L, 105), lambda i: (i, 0, 0)),
      out_shape=jax.ShapeDtypeStruct((B, L, 105), jnp.float32),
      interpret=interpret,
  )


def kernel(word_table, pos1_table, pos2_table, W, b, word,
           h_entity_word, t_entity_word, pos1, pos2):
  widx = word.reshape(N).astype(jnp.int32)
  hidx = h_entity_word.reshape(B).astype(jnp.int32)
  tidx = t_entity_word.reshape(B).astype(jnp.int32)
  wt = jnp.pad(word_table, ((0, 0), (0, WD - D)))
  we, he, te = _make_sc_gather()(wt, widx, hidx, tidx)
  wa = W[50:105, 0:50]
  wb = W[50:105, 50:100]
  w3 = W[50:105, 100:105]
  bs = jnp.broadcast_to(b[50:105], (K, GD))
  return _make_tc_dense()(we, he, te,
                          pos1.astype(jnp.int32), pos2.astype(jnp.int32),
                          pos1_table, pos2_table, wa, wb, w3, bs)


# speedup vs baseline: 3.2260x; 1.0396x over previous
import jax
import jax.numpy as jnp
from jax import lax
from jax.experimental import pallas as pl
from jax.experimental.pallas import tpu as pltpu
from jax.experimental.pallas import tpu_sc as plsc

B, L = 1024, 120
V, D = 100000, 50
P, PD = 512, 5
WD = 128
N = B * L
NC, NS = 2, 16
NW = NC * NS
PER_W = N // NW
CHUNK = 480
NCHUNK = PER_W // CHUNK
EPW = B // NW

GD = 55
K = 16
R = K * L


def _sc_gather_body(table, widx, hidx, tidx, we_out, he_out, te_out,
                    idx_v, rows_v, eidx_v, erows_v, sem):
  wid = lax.axis_index("s") * NC + lax.axis_index("c")
  base = wid * PER_W
  for c in range(NCHUNK):
    off = base + c * CHUNK
    pltpu.sync_copy(widx.at[pl.ds(off, CHUNK)], idx_v)
    pltpu.async_copy(table.at[idx_v], rows_v, sem).wait()
    pltpu.sync_copy(rows_v, we_out.at[pl.ds(off, CHUNK)])
  ebase = wid * EPW
  pltpu.sync_copy(hidx.at[pl.ds(ebase, EPW)], eidx_v)
  pltpu.async_copy(table.at[eidx_v], erows_v, sem).wait()
  pltpu.sync_copy(erows_v, he_out.at[pl.ds(ebase, EPW)])
  pltpu.sync_copy(tidx.at[pl.ds(ebase, EPW)], eidx_v)
  pltpu.async_copy(table.at[eidx_v], erows_v, sem).wait()
  pltpu.sync_copy(erows_v, te_out.at[pl.ds(ebase, EPW)])


def _make_sc_gather(interpret=False):
  mesh = plsc.VectorSubcoreMesh(core_axis_name="c", subcore_axis_name="s",
                                num_cores=NC, num_subcores=NS)
  return pl.kernel(
      _sc_gather_body,
      out_type=[jax.ShapeDtypeStruct((N, WD), jnp.float32),
                jax.ShapeDtypeStruct((B, WD), jnp.float32),
                jax.ShapeDtypeStruct((B, WD), jnp.float32)],
      mesh=mesh,
      scratch_types=[pltpu.VMEM((CHUNK,), jnp.int32),
                     pltpu.VMEM((CHUNK, WD), jnp.float32),
                     pltpu.VMEM((EPW,), jnp.int32),
                     pltpu.VMEM((EPW, WD), jnp.float32),
                     pltpu.SemaphoreType.DMA],
      compiler_params=pltpu.CompilerParams(use_tc_tiling_on_sc=True),
      interpret=interpret,
  )


def _dense_body(we_ref, he_ref, te_ref, p1_ref, p2_ref,
                tab1_ref, tab2_ref, wa_ref, wb_ref, w3_ref, bs_ref, out_ref):
  we = we_ref[...][:, 0:D]
  he = he_ref[...][:, 0:D]
  te = te_ref[...][:, 0:D]
  p1 = p1_ref[...][:, :, None]
  p2 = p2_ref[...][:, :, None]
  iot = lax.broadcasted_iota(jnp.int32, (1, 1, P), 2)
  oh1 = (p1 == iot).astype(jnp.float32).reshape(R, P)
  oh2 = (p2 == iot).astype(jnp.float32).reshape(R, P)
  tab1 = tab1_ref[...]
  tab2 = tab2_ref[...]
  w3 = w3_ref[...]
  t1w = lax.dot_general(tab1, w3, (((1,), (1,)), ((), ())),
                        preferred_element_type=jnp.float32)
  cat1 = jnp.concatenate([tab1, t1w], axis=1)
  r1 = lax.dot_general(oh1, cat1, (((1,), (0,)), ((), ())),
                       preferred_element_type=jnp.float32)
  p1v = r1[:, 0:PD]
  p1l = r1[:, PD:PD + GD]
  p2v = lax.dot_general(oh2, tab2, (((1,), (0,)), ((), ())),
                        preferred_element_type=jnp.float32)
  wl = lax.dot_general(we, wa_ref[...], (((1,), (1,)), ((), ())),
                       preferred_element_type=jnp.float32)
  hl = lax.dot_general(he, wb_ref[...], (((1,), (1,)), ((), ())),
                       preferred_element_type=jnp.float32)
  hl = hl + bs_ref[...]
  hlb = jnp.broadcast_to(hl[:, None, :], (K, L, GD)).reshape(R, GD)
  a = 1.0 / (1.0 + jnp.exp(-(wl + p1l + hlb)))
  amid = a[:, 0:D]
  ahi = a[:, D:GD]
  heb = jnp.broadcast_to(he[:, None, :], (K, L, D)).reshape(R, D)
  teb = jnp.broadcast_to(te[:, None, :], (K, L, D)).reshape(R, D)
  mid = teb + amid * (heb - teb)
  hi = p2v + ahi * (p1v - p2v)
  out = jnp.concatenate([we, mid, hi], axis=1)
  out_ref[...] = out.reshape(K, L, 105)


def _make_tc_dense(interpret=False):
  return pl.pallas_call(
      _dense_body,
      grid=(B // K,),
      in_specs=[
          pl.BlockSpec((R, WD), lambda i: (i, 0)),
          pl.BlockSpec((K, WD), lambda i: (i, 0)),
          pl.BlockSpec((K, WD), lambda i: (i, 0)),
          pl.BlockSpec((K, L), lambda i: (i, 0)),
          pl.BlockSpec((K, L), lambda i: (i, 0)),
          pl.BlockSpec((P, PD), lambda i: (0, 0)),
          pl.BlockSpec((P, PD), lambda i: (0, 0)),
          pl.BlockSpec((GD, D), lambda i: (0, 0)),
          pl.BlockSpec((GD, D), lambda i: (0, 0)),
          pl.BlockSpec((GD, PD), lambda i: (0, 0)),
          pl.BlockSpec((K, GD), lambda i: (0, 0)),
      ],
      out_specs=pl.BlockSpec((K, L, 105), lambda i: (i, 0, 0)),
      out_shape=jax.ShapeDtypeStruct((B, L, 105), jnp.float32),
      interpret=interpret,
  )


def kernel(word_table, pos1_table, pos2_table, W, b, word,
           h_entity_word, t_entity_word, pos1, pos2):
  widx = word.reshape(N).astype(jnp.int32)
  hidx = h_entity_word.reshape(B).astype(jnp.int32)
  tidx = t_entity_word.reshape(B).astype(jnp.int32)
  wt = jnp.pad(word_table, ((0, 0), (0, WD - D)))
  we, he, te = _make_sc_gather()(wt, widx, hidx, tidx)
  wa = W[50:105, 0:50]
  wb = W[50:105, 50:100]
  w3 = W[50:105, 100:105]
  bs = jnp.broadcast_to(b[50:105], (K, GD))
  return _make_tc_dense()(we, he, te,
                          pos1.astype(jnp.int32), pos2.astype(jnp.int32),
                          pos1_table, pos2_table, wa, wb, w3, bs)


# speedup vs baseline: 3.2462x; 1.0063x over previous
import jax
import jax.numpy as jnp
from jax import lax
from jax.experimental import pallas as pl
from jax.experimental.pallas import tpu as pltpu
from jax.experimental.pallas import tpu_sc as plsc

B, L = 1024, 120
V, D = 100000, 50
P, PD = 512, 5
WD = 128
PDW = 8
PC1 = 56
PC2 = 64
N = B * L
NC, NS = 2, 16
NW = NC * NS
PER_W = N // NW
CHUNK = 640
NCHUNK = PER_W // CHUNK
EPW = B // NW
LANES = 16

GD = 55
K = 16
R = K * L


def _sc_gather_body(table, widx, hidx, tidx, ptab1, ptab2, p1idx, p2idx,
                    we_out, he_out, te_out,
                    idx_v, rows_v, eidx_v, erows_v,
                    t1_v, t2_v, q1_v, q2_v, sem):
  wid = lax.axis_index("s") * NC + lax.axis_index("c")
  base = wid * PER_W
  pltpu.sync_copy(ptab1, t1_v)
  pltpu.sync_copy(ptab2, t2_v)
  lane = lax.broadcasted_iota(jnp.int32, (LANES,), 0)
  def chunk_body(c, carry):
    off = base + c * CHUNK
    pltpu.sync_copy(widx.at[pl.ds(off, CHUNK)], idx_v)
    pltpu.sync_copy(p1idx.at[pl.ds(off, CHUNK)], q1_v)
    pltpu.sync_copy(p2idx.at[pl.ds(off, CHUNK)], q2_v)
    pltpu.async_copy(table.at[idx_v], rows_v, sem).wait()

    def splice(j, carry2):
      tok = j * LANES + lane
      r1 = q1_v[pl.ds(j * LANES, LANES)]
      r2 = q2_v[pl.ds(j * LANES, LANES)]
      a1 = lax.shift_right_logical(r1, 4)
      b1 = lax.shift_left(jnp.bitwise_and(r1, 15), 3)
      a2 = lax.shift_right_logical(r2, 4)
      b2 = lax.shift_left(jnp.bitwise_and(r2, 15), 3)
      for col in range(PDW):
        colv = jnp.full((LANES,), col, jnp.int32)
        v1 = plsc.load_gather(t1_v, [a1, b1 + colv])
        plsc.store_scatter(rows_v, [tok, colv + PC1], v1)
        v2 = plsc.load_gather(t2_v, [a2, b2 + colv])
        plsc.store_scatter(rows_v, [tok, colv + PC2], v2)
      return carry2

    lax.fori_loop(0, CHUNK // LANES, splice, 0)
    pltpu.sync_copy(rows_v, we_out.at[pl.ds(off, CHUNK)])
    return carry

  lax.fori_loop(0, NCHUNK, chunk_body, 0)
  ebase = wid * EPW
  pltpu.sync_copy(hidx.at[pl.ds(ebase, EPW)], eidx_v)
  pltpu.async_copy(table.at[eidx_v], erows_v, sem).wait()
  pltpu.sync_copy(erows_v, he_out.at[pl.ds(ebase, EPW)])
  pltpu.sync_copy(tidx.at[pl.ds(ebase, EPW)], eidx_v)
  pltpu.async_copy(table.at[eidx_v], erows_v, sem).wait()
  pltpu.sync_copy(erows_v, te_out.at[pl.ds(ebase, EPW)])


def _make_sc_gather(interpret=False):
  mesh = plsc.VectorSubcoreMesh(core_axis_name="c", subcore_axis_name="s",
                                num_cores=NC, num_subcores=NS)
  return pl.kernel(
      _sc_gather_body,
      out_type=[jax.ShapeDtypeStruct((N, WD), jnp.float32),
                jax.ShapeDtypeStruct((B, WD), jnp.float32),
                jax.ShapeDtypeStruct((B, WD), jnp.float32)],
      mesh=mesh,
      scratch_types=[pltpu.VMEM((CHUNK,), jnp.int32),
                     pltpu.VMEM((CHUNK, WD), jnp.float32),
                     pltpu.VMEM((EPW,), jnp.int32),
                     pltpu.VMEM((EPW, WD), jnp.float32),
                     pltpu.VMEM((P * PDW // 128, 128), jnp.float32),
                     pltpu.VMEM((P * PDW // 128, 128), jnp.float32),
                     pltpu.VMEM((CHUNK,), jnp.int32),
                     pltpu.VMEM((CHUNK,), jnp.int32),
                     pltpu.SemaphoreType.DMA],
      compiler_params=pltpu.CompilerParams(use_tc_tiling_on_sc=True,
                                           needs_layout_passes=False),
      interpret=interpret,
  )


def _dense_body(we_ref, he_ref, te_ref, wa_ref, wb_ref, w3_ref, bs_ref,
                out_ref):
  blk = we_ref[...]
  we = blk[:, 0:D]
  p1r = blk[:, PC1:PC1 + PDW]
  p2r = blk[:, PC2:PC2 + PDW]
  he = he_ref[...][:, 0:D]
  te = te_ref[...][:, 0:D]
  wl = lax.dot_general(we, wa_ref[...], (((1,), (1,)), ((), ())),
                       preferred_element_type=jnp.float32)
  p1l = lax.dot_general(p1r, w3_ref[...], (((1,), (1,)), ((), ())),
                        preferred_element_type=jnp.float32)
  hl = lax.dot_general(he, wb_ref[...], (((1,), (1,)), ((), ())),
                       preferred_element_type=jnp.float32)
  hl = hl + bs_ref[...]
  hlb = jnp.broadcast_to(hl[:, None, :], (K, L, GD)).reshape(R, GD)
  a = 1.0 / (1.0 + jnp.exp(-(wl + p1l + hlb)))
  amid = a[:, 0:D]
  ahi = a[:, D:GD]
  heb = jnp.broadcast_to(he[:, None, :], (K, L, D)).reshape(R, D)
  teb = jnp.broadcast_to(te[:, None, :], (K, L, D)).reshape(R, D)
  mid = teb + amid * (heb - teb)
  p1v = p1r[:, 0:PD]
  p2v = p2r[:, 0:PD]
  hi = p2v + ahi * (p1v - p2v)
  out = jnp.concatenate([we, mid, hi], axis=1)
  out_ref[...] = out.reshape(K, L, 105)


def _make_tc_dense(interpret=False):
  return pl.pallas_call(
      _dense_body,
      grid=(B // K,),
      in_specs=[
          pl.BlockSpec((R, WD), lambda i: (i, 0)),
          pl.BlockSpec((K, WD), lambda i: (i, 0)),
          pl.BlockSpec((K, WD), lambda i: (i, 0)),
          pl.BlockSpec((GD, D), lambda i: (0, 0)),
          pl.BlockSpec((GD, D), lambda i: (0, 0)),
          pl.BlockSpec((GD, PDW), lambda i: (0, 0)),
          pl.BlockSpec((K, GD), lambda i: (0, 0)),
      ],
      out_specs=pl.BlockSpec((K, L, 105), lambda i: (i, 0, 0)),
      out_shape=jax.ShapeDtypeStruct((B, L, 105), jnp.float32),
      interpret=interpret,
  )


def kernel(word_table, pos1_table, pos2_table, W, b, word,
           h_entity_word, t_entity_word, pos1, pos2):
  widx = word.reshape(N).astype(jnp.int32)
  hidx = h_entity_word.reshape(B).astype(jnp.int32)
  tidx = t_entity_word.reshape(B).astype(jnp.int32)
  p1idx = pos1.reshape(N).astype(jnp.int32)
  p2idx = pos2.reshape(N).astype(jnp.int32)
  wt = jnp.pad(word_table, ((0, 0), (0, WD - D)))
  ptab1 = jnp.pad(pos1_table, ((0, 0), (0, PDW - PD))).reshape(P * PDW // 128, 128)
  ptab2 = jnp.pad(pos2_table, ((0, 0), (0, PDW - PD))).reshape(P * PDW // 128, 128)
  we, he, te = _make_sc_gather()(wt, widx, hidx, tidx,
                                 ptab1, ptab2, p1idx, p2idx)
  wa = W[50:105, 0:50]
  wb = W[50:105, 50:100]
  w3 = jnp.pad(W[50:105, 100:105], ((0, 0), (0, PDW - PD)))
  bs = jnp.broadcast_to(b[50:105], (K, GD))
  return _make_tc_dense()(we, he, te, wa, wb, w3, bs)


# speedup vs baseline: 3.2980x; 1.0159x over previous
import jax
import jax.numpy as jnp
from jax import lax
from jax.experimental import pallas as pl
from jax.experimental.pallas import tpu as pltpu
from jax.experimental.pallas import tpu_sc as plsc

B, L = 1024, 120
V, D = 100000, 50
P, PD = 512, 5
WD = 128
PDW = 8
PC1 = 56
PC2 = 64
N = B * L
NC, NS = 2, 16
NW = NC * NS
PER_W = N // NW
CHUNK = 640
NCHUNK = PER_W // CHUNK
EPW = B // NW
LANES = 16

GD = 55
K = 32
R = K * L


def _sc_gather_body(table, widx, hidx, tidx, ptab1, ptab2, p1idx, p2idx,
                    we_out, he_out, te_out,
                    idx_v, rows_v, eidx_v, erows_v,
                    t1_v, t2_v, q1_v, q2_v, sem):
  wid = lax.axis_index("s") * NC + lax.axis_index("c")
  base = wid * PER_W
  pltpu.sync_copy(ptab1, t1_v)
  pltpu.sync_copy(ptab2, t2_v)
  lane = lax.broadcasted_iota(jnp.int32, (LANES,), 0)
  def chunk_body(c, carry):
    off = base + c * CHUNK
    pltpu.sync_copy(widx.at[pl.ds(off, CHUNK)], idx_v)
    pltpu.sync_copy(p1idx.at[pl.ds(off, CHUNK)], q1_v)
    pltpu.sync_copy(p2idx.at[pl.ds(off, CHUNK)], q2_v)
    pltpu.async_copy(table.at[idx_v], rows_v, sem).wait()

    def splice(j, carry2):
      tok = j * LANES + lane
      r1 = q1_v[pl.ds(j * LANES, LANES)]
      r2 = q2_v[pl.ds(j * LANES, LANES)]
      a1 = lax.shift_right_logical(r1, 4)
      b1 = lax.shift_left(jnp.bitwise_and(r1, 15), 3)
      a2 = lax.shift_right_logical(r2, 4)
      b2 = lax.shift_left(jnp.bitwise_and(r2, 15), 3)
      for col in range(PDW):
        colv = jnp.full((LANES,), col, jnp.int32)
        v1 = plsc.load_gather(t1_v, [a1, b1 + colv])
        plsc.store_scatter(rows_v, [tok, colv + PC1], v1)
        v2 = plsc.load_gather(t2_v, [a2, b2 + colv])
        plsc.store_scatter(rows_v, [tok, colv + PC2], v2)
      return carry2

    lax.fori_loop(0, CHUNK // LANES, splice, 0)
    pltpu.sync_copy(rows_v, we_out.at[pl.ds(off, CHUNK)])
    return carry

  lax.fori_loop(0, NCHUNK, chunk_body, 0)
  ebase = wid * EPW
  pltpu.sync_copy(hidx.at[pl.ds(ebase, EPW)], eidx_v)
  pltpu.async_copy(table.at[eidx_v], erows_v, sem).wait()
  pltpu.sync_copy(erows_v, he_out.at[pl.ds(ebase, EPW)])
  pltpu.sync_copy(tidx.at[pl.ds(ebase, EPW)], eidx_v)
  pltpu.async_copy(table.at[eidx_v], erows_v, sem).wait()
  pltpu.sync_copy(erows_v, te_out.at[pl.ds(ebase, EPW)])


def _make_sc_gather(interpret=False):
  mesh = plsc.VectorSubcoreMesh(core_axis_name="c", subcore_axis_name="s",
                                num_cores=NC, num_subcores=NS)
  return pl.kernel(
      _sc_gather_body,
      out_type=[jax.ShapeDtypeStruct((N, WD), jnp.float32),
                jax.ShapeDtypeStruct((B, WD), jnp.float32),
                jax.ShapeDtypeStruct((B, WD), jnp.float32)],
      mesh=mesh,
      scratch_types=[pltpu.VMEM((CHUNK,), jnp.int32),
                     pltpu.VMEM((CHUNK, WD), jnp.float32),
                     pltpu.VMEM((EPW,), jnp.int32),
                     pltpu.VMEM((EPW, WD), jnp.float32),
                     pltpu.VMEM((P * PDW // 128, 128), jnp.float32),
                     pltpu.VMEM((P * PDW // 128, 128), jnp.float32),
                     pltpu.VMEM((CHUNK,), jnp.int32),
                     pltpu.VMEM((CHUNK,), jnp.int32),
                     pltpu.SemaphoreType.DMA],
      compiler_params=pltpu.CompilerParams(use_tc_tiling_on_sc=True,
                                           needs_layout_passes=False),
      interpret=interpret,
  )


def _dense_body(we_ref, he_ref, te_ref, wa_ref, wb_ref, w3_ref, bs_ref,
                out_ref):
  blk = we_ref[...]
  we = blk[:, 0:D]
  p1r = blk[:, PC1:PC1 + PDW]
  p2r = blk[:, PC2:PC2 + PDW]
  he = he_ref[...][:, 0:D]
  te = te_ref[...][:, 0:D]
  wl = lax.dot_general(we, wa_ref[...], (((1,), (1,)), ((), ())),
                       preferred_element_type=jnp.float32)
  p1l = lax.dot_general(p1r, w3_ref[...], (((1,), (1,)), ((), ())),
                        preferred_element_type=jnp.float32)
  hl = lax.dot_general(he, wb_ref[...], (((1,), (1,)), ((), ())),
                       preferred_element_type=jnp.float32)
  hl = hl + bs_ref[...]
  hlb = jnp.broadcast_to(hl[:, None, :], (K, L, GD)).reshape(R, GD)
  a = 1.0 / (1.0 + jnp.exp(-(wl + p1l + hlb)))
  amid = a[:, 0:D]
  ahi = a[:, D:GD]
  heb = jnp.broadcast_to(he[:, None, :], (K, L, D)).reshape(R, D)
  teb = jnp.broadcast_to(te[:, None, :], (K, L, D)).reshape(R, D)
  mid = teb + amid * (heb - teb)
  p1v = p1r[:, 0:PD]
  p2v = p2r[:, 0:PD]
  hi = p2v + ahi * (p1v - p2v)
  out = jnp.concatenate([we, mid, hi], axis=1)
  out_ref[...] = out.reshape(K, L, 105)


def _make_tc_dense(interpret=False):
  return pl.pallas_call(
      _dense_body,
      grid=(B // K,),
      in_specs=[
          pl.BlockSpec((R, WD), lambda i: (i, 0)),
          pl.BlockSpec((K, WD), lambda i: (i, 0)),
          pl.BlockSpec((K, WD), lambda i: (i, 0)),
          pl.BlockSpec((GD, D), lambda i: (0, 0)),
          pl.BlockSpec((GD, D), lambda i: (0, 0)),
          pl.BlockSpec((GD, PDW), lambda i: (0, 0)),
          pl.BlockSpec((K, GD), lambda i: (0, 0)),
      ],
      out_specs=pl.BlockSpec((K, L, 105), lambda i: (i, 0, 0)),
      out_shape=jax.ShapeDtypeStruct((B, L, 105), jnp.float32),
      interpret=interpret,
  )


def kernel(word_table, pos1_table, pos2_table, W, b, word,
           h_entity_word, t_entity_word, pos1, pos2):
  widx = word.reshape(N).astype(jnp.int32)
  hidx = h_entity_word.reshape(B).astype(jnp.int32)
  tidx = t_entity_word.reshape(B).astype(jnp.int32)
  p1idx = pos1.reshape(N).astype(jnp.int32)
  p2idx = pos2.reshape(N).astype(jnp.int32)
  wt = jnp.pad(word_table, ((0, 0), (0, WD - D)))
  ptab1 = jnp.pad(pos1_table, ((0, 0), (0, PDW - PD))).reshape(P * PDW // 128, 128)
  ptab2 = jnp.pad(pos2_table, ((0, 0), (0, PDW - PD))).reshape(P * PDW // 128, 128)
  we, he, te = _make_sc_gather()(wt, widx, hidx, tidx,
                                 ptab1, ptab2, p1idx, p2idx)
  wa = W[50:105, 0:50]
  wb = W[50:105, 50:100]
  w3 = jnp.pad(W[50:105, 100:105], ((0, 0), (0, PDW - PD)))
  bs = jnp.broadcast_to(b[50:105], (K, GD))
  return _make_tc_dense()(we, he, te, wa, wb, w3, bs)


# speedup vs baseline: 3.3006x; 1.0008x over previous
import jax
import jax.numpy as jnp
from jax import lax
from jax.experimental import pallas as pl
from jax.experimental.pallas import tpu as pltpu
from jax.experimental.pallas import tpu_sc as plsc

B, L = 1024, 120
V, D = 100000, 50
P, PD = 512, 5
WD = 128
PDW = 8
PC1 = 56
PC2 = 64
N = B * L
NC, NS = 2, 16
NW = NC * NS
PER_W = N // NW
CHUNK = 640
NCHUNK = PER_W // CHUNK
EPW = B // NW
LANES = 16

GD = 55
K = 32
R = K * L


def _sc_gather_body(table, widx, hidx, tidx, ptab1, ptab2, p1idx, p2idx,
                    we_out, he_out, te_out,
                    idx_v, rows_v, eidx_v, erows_v,
                    t1_v, t2_v, q1_v, q2_v, sem):
  wid = lax.axis_index("s") * NC + lax.axis_index("c")
  base = wid * PER_W
  pltpu.sync_copy(ptab1, t1_v)
  pltpu.sync_copy(ptab2, t2_v)
  lane = lax.broadcasted_iota(jnp.int32, (LANES,), 0)
  def chunk_body(c, carry):
    off = base + c * CHUNK
    pltpu.sync_copy(widx.at[pl.ds(off, CHUNK)], idx_v)
    pltpu.sync_copy(p1idx.at[pl.ds(off, CHUNK)], q1_v)
    pltpu.sync_copy(p2idx.at[pl.ds(off, CHUNK)], q2_v)
    pltpu.async_copy(table.at[idx_v], rows_v, sem).wait()

    def splice(j, carry2):
      tok = j * LANES + lane
      r1 = q1_v[pl.ds(j * LANES, LANES)]
      r2 = q2_v[pl.ds(j * LANES, LANES)]
      a1 = lax.shift_right_logical(r1, 4)
      b1 = lax.shift_left(jnp.bitwise_and(r1, 15), 3)
      a2 = lax.shift_right_logical(r2, 4)
      b2 = lax.shift_left(jnp.bitwise_and(r2, 15), 3)
      for col in range(PDW):
        colv = jnp.full((LANES,), col, jnp.int32)
        v1 = plsc.load_gather(t1_v, [a1, b1 + colv])
        plsc.store_scatter(rows_v, [tok, colv + PC1], v1)
        v2 = plsc.load_gather(t2_v, [a2, b2 + colv])
        plsc.store_scatter(rows_v, [tok, colv + PC2], v2)
      return carry2

    lax.fori_loop(0, CHUNK // LANES, splice, 0)
    pltpu.sync_copy(rows_v, we_out.at[pl.ds(off, CHUNK)])
    return carry

  lax.fori_loop(0, NCHUNK, chunk_body, 0)
  ebase = wid * EPW
  pltpu.sync_copy(hidx.at[pl.ds(ebase, EPW)], eidx_v)
  pltpu.async_copy(table.at[eidx_v], erows_v, sem).wait()
  pltpu.sync_copy(erows_v, he_out.at[pl.ds(ebase, EPW)])
  pltpu.sync_copy(tidx.at[pl.ds(ebase, EPW)], eidx_v)
  pltpu.async_copy(table.at[eidx_v], erows_v, sem).wait()
  pltpu.sync_copy(erows_v, te_out.at[pl.ds(ebase, EPW)])


def _make_sc_gather():
  mesh = plsc.VectorSubcoreMesh(core_axis_name="c", subcore_axis_name="s",
                                num_cores=NC, num_subcores=NS)
  return pl.kernel(
      _sc_gather_body,
      out_type=[jax.ShapeDtypeStruct((N, WD), jnp.float32),
                jax.ShapeDtypeStruct((B, WD), jnp.float32),
                jax.ShapeDtypeStruct((B, WD), jnp.float32)],
      mesh=mesh,
      scratch_types=[pltpu.VMEM((CHUNK,), jnp.int32),
                     pltpu.VMEM((CHUNK, WD), jnp.float32),
                     pltpu.VMEM((EPW,), jnp.int32),
                     pltpu.VMEM((EPW, WD), jnp.float32),
                     pltpu.VMEM((P * PDW // 128, 128), jnp.float32),
                     pltpu.VMEM((P * PDW // 128, 128), jnp.float32),
                     pltpu.VMEM((CHUNK,), jnp.int32),
                     pltpu.VMEM((CHUNK,), jnp.int32),
                     pltpu.SemaphoreType.DMA],
      compiler_params=pltpu.CompilerParams(use_tc_tiling_on_sc=True,
                                           needs_layout_passes=False),
  )


def _dense_body(we_ref, he_ref, te_ref, wa_ref, wb_ref, w3_ref, bs_ref,
                out_ref):
  blk = we_ref[...]
  we = blk[:, 0:D]
  p1r = blk[:, PC1:PC1 + PDW]
  p2r = blk[:, PC2:PC2 + PDW]
  he = he_ref[...][:, 0:D]
  te = te_ref[...][:, 0:D]
  wl = lax.dot_general(we, wa_ref[...], (((1,), (1,)), ((), ())),
                       preferred_element_type=jnp.float32)
  p1l = lax.dot_general(p1r, w3_ref[...], (((1,), (1,)), ((), ())),
                        preferred_element_type=jnp.float32)
  hl = lax.dot_general(he, wb_ref[...], (((1,), (1,)), ((), ())),
                       preferred_element_type=jnp.float32)
  hl = hl + bs_ref[...]
  hlb = jnp.broadcast_to(hl[:, None, :], (K, L, GD)).reshape(R, GD)
  a = 1.0 / (1.0 + jnp.exp(-(wl + p1l + hlb)))
  amid = a[:, 0:D]
  ahi = a[:, D:GD]
  heb = jnp.broadcast_to(he[:, None, :], (K, L, D)).reshape(R, D)
  teb = jnp.broadcast_to(te[:, None, :], (K, L, D)).reshape(R, D)
  mid = teb + amid * (heb - teb)
  p1v = p1r[:, 0:PD]
  p2v = p2r[:, 0:PD]
  hi = p2v + ahi * (p1v - p2v)
  out = jnp.concatenate([we, mid, hi], axis=1)
  out_ref[...] = out.reshape(K, L, 105)


def _make_tc_dense():
  return pl.pallas_call(
      _dense_body,
      grid=(B // K,),
      in_specs=[
          pl.BlockSpec((R, WD), lambda i: (i, 0)),
          pl.BlockSpec((K, WD), lambda i: (i, 0)),
          pl.BlockSpec((K, WD), lambda i: (i, 0)),
          pl.BlockSpec((GD, D), lambda i: (0, 0)),
          pl.BlockSpec((GD, D), lambda i: (0, 0)),
          pl.BlockSpec((GD, PDW), lambda i: (0, 0)),
          pl.BlockSpec((K, GD), lambda i: (0, 0)),
      ],
      out_specs=pl.BlockSpec((K, L, 105), lambda i: (i, 0, 0)),
      out_shape=jax.ShapeDtypeStruct((B, L, 105), jnp.float32),
  )


def kernel(word_table, pos1_table, pos2_table, W, b, word,
           h_entity_word, t_entity_word, pos1, pos2):
  widx = word.reshape(N).astype(jnp.int32)
  hidx = h_entity_word.reshape(B).astype(jnp.int32)
  tidx = t_entity_word.reshape(B).astype(jnp.int32)
  p1idx = pos1.reshape(N).astype(jnp.int32)
  p2idx = pos2.reshape(N).astype(jnp.int32)
  wt = jnp.pad(word_table, ((0, 0), (0, WD - D)))
  ptab1 = jnp.pad(pos1_table, ((0, 0), (0, PDW - PD))).reshape(P * PDW // 128, 128)
  ptab2 = jnp.pad(pos2_table, ((0, 0), (0, PDW - PD))).reshape(P * PDW // 128, 128)
  we, he, te = _make_sc_gather()(wt, widx, hidx, tidx,
                                 ptab1, ptab2, p1idx, p2idx)
  wa = W[50:105, 0:50]
  wb = W[50:105, 50:100]
  w3 = jnp.pad(W[50:105, 100:105], ((0, 0), (0, PDW - PD)))
  bs = jnp.broadcast_to(b[50:105], (K, GD))
  return _make_tc_dense()(we, he, te, wa, wb, w3, bs)
